# baseline (speedup 1.0000x reference)
import functools

import jax
import jax.numpy as jnp
from jax import lax
from jax.experimental import pallas as pl
from jax.experimental.pallas import tpu as pltpu
from jax.experimental.pallas import tpu_sc as plsc

_C = 19
_MS = 20000
_M = _C * _MS
_D = 128
_N = 16384

_NW = 32
_RW = _M // _NW
_PPAD = ((_RW + 15) // 16) * 16
_KC = 128
_NCH = _N // _KC + 1
_LCAP = _NCH * _KC
_WCAP = 8
_NB = 6

_BR = 20000
_NF = _D // 16


def _copy_body(m_ref, o_ref, s_ref):
    i = pl.program_id(0)
    blk = m_ref[...]
    o_ref[...] = blk
    part = blk.reshape(_BR // 8, 8, _D).sum(axis=0)

    @pl.when(i == 0)
    def _():
        s_ref[...] = part

    @pl.when(i > 0)
    def _():
        s_ref[...] = s_ref[...] + part


def _final_body(cs_ref, dn_ref, do_ref, o_ref):
    s = (jnp.sum(cs_ref[...], axis=0, keepdims=True)
         + jnp.sum(dn_ref[...], axis=0, keepdims=True)
         - jnp.sum(do_ref[...], axis=0, keepdims=True))
    t = s * (1.0 / _MS)
    o_ref[0, 0] = jnp.sum(t * t) + float(_C * (_C - 1))


def _normalize_row(buf, i):
    vs = [buf[i, pl.ds(f * 16, 16)] for f in range(_NF)]
    ss = vs[0] * vs[0]
    for f in range(1, _NF):
        ss = ss + vs[f] * vs[f]
    s = jnp.sum(ss)
    sv = jnp.full((16,), s, jnp.float32)
    bits = plsc.bitcast(sv, jnp.int32)
    y = plsc.bitcast(jnp.int32(0x5F3759DF) - lax.shift_right_arithmetic(bits, 1),
                     jnp.float32)
    for _ in range(3):
        y = y * (1.5 - 0.5 * sv * y * y)
    y = jnp.where(sv > 0.0, y, 0.0)
    ws = []
    for f in range(_NF):
        w = vs[f] * y
        buf[i, pl.ds(f * 16, 16)] = w
        ws.append(w)
    return ws


def _plan_body(idx_h, val_h, delta_h, rflat_h, jflat_h, ncnt_h, wdat_h,
               idx_v, pbuf, nb, rlist, jlist, stage_r, stage_j,
               vbuf, acc, nbuf, spidx, sem_a, sem_b):
    cid = lax.axis_index("c")
    sid = lax.axis_index("s")
    wid = sid * 2 + cid
    lo = wid * _RW

    @pl.when(sid == 0)
    def _():
        pltpu.sync_copy(idx_h, spidx)
    plsc.subcore_barrier()
    pltpu.sync_copy(spidx, idx_v)

    lane = lax.iota(jnp.int32, 16)
    neg1 = jnp.full((16,), -1, jnp.int32)
    zeros = jnp.zeros((16,), jnp.float32)

    def _init(i, _):
        pbuf[pl.ds(i * 16, 16)] = neg1
        return 0
    lax.fori_loop(0, _PPAD // 16, _init, 0)
    nb[pl.ds(16, 16)] = neg1

    def _p1(i, _):
        iv = idx_v[pl.ds(i * 16, 16)]
        key = iv * 16 + lane
        sk, _sv = plsc.sort_key_val(key, key)
        sidx = lax.shift_right_arithmetic(sk, 4)
        nb[pl.ds(0, 16)] = sidx
        nxt = nb[pl.ds(1, 16)]
        keep = jnp.not_equal(sidx, nxt)
        offs = sidx - lo
        inr = (offs >= 0) & (offs < _RW)
        m = keep & inr
        offs_c = jnp.where(m, offs, 0)
        jwin = (sk & 15) + i * 16
        plsc.store_scatter(pbuf, [offs_c], jwin, mask=m)
        return 0
    lax.fori_loop(0, _N // 16, _p1, 0)

    def _p2(k, o):
        pv = pbuf[pl.ds(k * 16, 16)]
        msk = pv >= 0
        rvec = (lo + k * 16) + lane
        plsc.store_compressed(rlist.at[pl.ds(o, 16)], rvec, mask=msk)
        plsc.store_compressed(jlist.at[pl.ds(o, 16)], pv, mask=msk)
        return o + jnp.sum(msk.astype(jnp.int32))
    n = lax.fori_loop(0, _PPAD // 16, _p2, 0)

    nbuf[pl.ds(0, 16)] = jnp.full((16,), n, jnp.int32)
    pltpu.sync_copy(nbuf, ncnt_h.at[wid])

    for f in range(_NF):
        acc[pl.ds(f * 16, 16)] = zeros

    @pl.when(n > 0)
    def _():
        r0 = rlist[pl.ds(0, 16)][0]
        j0 = jlist[pl.ds(0, 16)][0]
        r0v = jnp.full((16,), r0, jnp.int32)
        j0v = jnp.full((16,), j0, jnp.int32)
        for t in range(_KC // 16):
            rlist[pl.ds(n + t * 16, 16)] = r0v
            jlist[pl.ds(n + t * 16, 16)] = j0v
        nch = (n + _KC - 1) // _KC

        def _p3(c, accs):
            base = c * _KC
            for t in range(_KC // 16):
                stage_r[pl.ds(t * 16, 16)] = rlist[pl.ds(base + t * 16, 16)]
                stage_j[pl.ds(t * 16, 16)] = jlist[pl.ds(base + t * 16, 16)]
            d1 = pltpu.async_copy(val_h.at[stage_j], vbuf, sem_a)
            pltpu.sync_copy(stage_r, rflat_h.at[wid, c])
            pltpu.sync_copy(stage_j, jflat_h.at[wid, c])
            d1.wait()

            def _row(i, a):
                ws = _normalize_row(vbuf, i)
                return tuple(a[f] + ws[f] for f in range(_NF))
            accs = lax.fori_loop(0, _KC, _row, accs)

            @pl.when(c < _WCAP)
            def _():
                pltpu.sync_copy(vbuf, wdat_h.at[wid, c])
            return accs

        accs = lax.fori_loop(0, nch, _p3, (zeros,) * _NF)

        npad = nch * _KC - n
        w = npad.astype(jnp.float32)
        for f in range(_NF):
            acc[pl.ds(f * 16, 16)] = (
                accs[f] - vbuf[_KC - 1, pl.ds(f * 16, 16)] * w)

    pltpu.sync_copy(acc, delta_h.at[wid])


def _apply_body(val_h, mem_h, rflat_h, jflat_h, ncnt_h, wdat_h, out_h, dold_h,
                nbuf, stage_r, stage_j, vbuf, acc2, sem_a, sem_b):
    cid = lax.axis_index("c")
    sid = lax.axis_index("s")
    wid = sid * 2 + cid
    zeros = jnp.zeros((16,), jnp.float32)

    pltpu.async_copy(ncnt_h.at[wid], nbuf, sem_b)
    for b in range(_NB):
        pltpu.async_copy(rflat_h.at[wid, b], stage_r.at[b], sem_b)
        pltpu.async_copy(wdat_h.at[wid, b], vbuf.at[b], sem_a)
    pltpu.make_async_copy(ncnt_h.at[wid], nbuf, sem_b).wait()
    for b in range(_NB):
        pltpu.make_async_copy(rflat_h.at[wid, b], stage_r.at[b], sem_b).wait()
        pltpu.make_async_copy(wdat_h.at[wid, b], vbuf.at[b], sem_a).wait()
    n = nbuf[pl.ds(0, 16)][0]
    nch = (n + _KC - 1) // _KC

    for b in range(_NB):
        @pl.when(b < nch)
        def _():
            pltpu.async_copy(vbuf.at[b], out_h.at[stage_r.at[b]], sem_a)
    for b in range(_NB):
        @pl.when(b < nch)
        def _():
            pltpu.make_async_copy(vbuf.at[b], out_h.at[stage_r.at[b]],
                                  sem_a).wait()

    for b in range(_NB):
        @pl.when(b < nch)
        def _():
            pltpu.async_copy(mem_h.at[stage_r.at[b]], vbuf.at[b], sem_a)
    for b in range(_NB):
        @pl.when(b < nch)
        def _():
            pltpu.make_async_copy(mem_h.at[stage_r.at[b]], vbuf.at[b],
                                  sem_a).wait()

    for f in range(_NF):
        acc2[pl.ds(f * 16, 16)] = zeros

    @pl.when(n > 0)
    def _():
        accs = (zeros,) * _NF
        for b in range(_NB):
            def _rsum(i, a, b=b):
                return tuple(a[f] + vbuf[b, i, pl.ds(f * 16, 16)]
                             for f in range(_NF))
            ssum = lax.fori_loop(0, _KC, _rsum, (zeros,) * _NF)
            valid = b < nch
            accs = tuple(accs[f] + jnp.where(valid, ssum[f], zeros)
                         for f in range(_NF))

        def _tail(c, a):
            pltpu.sync_copy(rflat_h.at[wid, c], stage_r.at[0])

            @pl.when(c < _WCAP)
            def _():
                pltpu.sync_copy(wdat_h.at[wid, c], vbuf.at[0])

            @pl.when(c >= _WCAP)
            def _():
                pltpu.sync_copy(jflat_h.at[wid, c], stage_j.at[0])
                pltpu.async_copy(val_h.at[stage_j.at[0]], vbuf.at[0],
                                 sem_a).wait()

                def _row(i, _):
                    _normalize_row(vbuf.at[0], i)
                    return 0
                lax.fori_loop(0, _KC, _row, 0)

            pltpu.async_copy(vbuf.at[0], out_h.at[stage_r.at[0]],
                             sem_a).wait()
            pltpu.async_copy(mem_h.at[stage_r.at[0]], vbuf.at[0],
                             sem_a).wait()

            def _orow(i, aa):
                return tuple(aa[f] + vbuf[0, i, pl.ds(f * 16, 16)]
                             for f in range(_NF))
            return lax.fori_loop(0, _KC, _orow, a)
        accs = lax.fori_loop(_NB, nch, _tail, accs)

        npad = nch * _KC - n
        w = npad.astype(jnp.float32)
        b_pad = jnp.where(nch <= _NB, nch - 1, 0)
        for f in range(_NF):
            acc2[pl.ds(f * 16, 16)] = (
                accs[f] - vbuf[b_pad, _KC - 1, pl.ds(f * 16, 16)] * w)

    pltpu.sync_copy(acc2, dold_h.at[wid])


_copy_call = pl.pallas_call(
    _copy_body,
    grid=(_M // _BR,),
    in_specs=[pl.BlockSpec((_BR, _D), lambda i: (i, 0))],
    out_specs=[
        pl.BlockSpec((_BR, _D), lambda i: (i, 0)),
        pl.BlockSpec((8, _D), lambda i: (0, 0)),
    ],
    out_shape=[
        jax.ShapeDtypeStruct((_M, _D), jnp.float32),
        jax.ShapeDtypeStruct((8, _D), jnp.float32),
    ],
)

_final_call = pl.pallas_call(
    _final_body,
    out_specs=pl.BlockSpec(memory_space=pltpu.SMEM),
    out_shape=jax.ShapeDtypeStruct((1, 1), jnp.float32),
)


@functools.cache
def _get_plan_call():
  return pl.kernel(
    _plan_body,
    out_type=(
        jax.ShapeDtypeStruct((_NW, _D), jnp.float32),
        jax.ShapeDtypeStruct((_NW, _NCH, _KC), jnp.int32),
        jax.ShapeDtypeStruct((_NW, _NCH, _KC), jnp.int32),
        jax.ShapeDtypeStruct((_NW, 16), jnp.int32),
        jax.ShapeDtypeStruct((_NW, _WCAP, _KC, _D), jnp.float32),
    ),
    mesh=plsc.VectorSubcoreMesh(core_axis_name="c", subcore_axis_name="s"),
    compiler_params=pltpu.CompilerParams(needs_layout_passes=False),
    scratch_types=[
        pltpu.VMEM((_N,), jnp.int32),
        pltpu.VMEM((_PPAD,), jnp.int32),
        pltpu.VMEM((32,), jnp.int32),
        pltpu.VMEM((_LCAP,), jnp.int32),
        pltpu.VMEM((_LCAP,), jnp.int32),
        pltpu.VMEM((_KC,), jnp.int32),
        pltpu.VMEM((_KC,), jnp.int32),
        pltpu.VMEM((_KC, _D), jnp.float32),
        pltpu.VMEM((_D,), jnp.float32),
        pltpu.VMEM((16,), jnp.int32),
        pltpu.VMEM_SHARED((_N,), jnp.int32),
        pltpu.SemaphoreType.DMA,
        pltpu.SemaphoreType.DMA,
    ],
  )


@functools.cache
def _get_apply_call():
  return pl.kernel(
    _apply_body,
    out_type=jax.ShapeDtypeStruct((_NW, _D), jnp.float32),
    mesh=plsc.VectorSubcoreMesh(core_axis_name="c", subcore_axis_name="s"),
    compiler_params=pltpu.CompilerParams(needs_layout_passes=False),
    scratch_types=[
        pltpu.VMEM((16,), jnp.int32),
        pltpu.VMEM((_NB, _KC), jnp.int32),
        pltpu.VMEM((_NB, _KC), jnp.int32),
        pltpu.VMEM((_NB, _KC, _D), jnp.float32),
        pltpu.VMEM((_D,), jnp.float32),
        pltpu.SemaphoreType.DMA,
        pltpu.SemaphoreType.DMA,
    ],
  )


@jax.jit
def kernel(mem, idx, val):
    idx = idx.astype(jnp.int32)
    delta_new, rflat, jflat, ncnt, wdat = _get_plan_call()(idx, val)
    out_copy, colsum8 = _copy_call(mem)
    out_ref = jax.new_ref(out_copy)
    delta_old = _get_apply_call()(val, mem, rflat, jflat, ncnt, wdat, out_ref)
    ratio_arr = _final_call(colsum8, delta_new, delta_old)
    out_final = jax.freeze(out_ref)
    return (out_final, ratio_arr[0, 0])


# speedup vs baseline: 1.0670x; 1.0670x over previous
import functools

import jax
import jax.numpy as jnp
from jax import lax
from jax.experimental import pallas as pl
from jax.experimental.pallas import tpu as pltpu
from jax.experimental.pallas import tpu_sc as plsc

_C = 19
_MS = 20000
_M = _C * _MS
_D = 128
_N = 16384

_NW = 32
_RW = _M // _NW
_PPAD = ((_RW + 15) // 16) * 16
_KC = 128
_NCH = _N // _KC + 1
_LCAP = _NCH * _KC
_WCAP = 8
_NB = 6

_BR = 20000
_NF = _D // 16


def _copy_body(m_ref, o_ref, s_ref):
    i = pl.program_id(0)
    blk = m_ref[...]
    o_ref[...] = blk
    part = blk.reshape(_BR // 8, 8, _D).sum(axis=0)

    @pl.when(i == 0)
    def _():
        s_ref[...] = part

    @pl.when(i > 0)
    def _():
        s_ref[...] = s_ref[...] + part


def _final_body(cs_ref, dl_ref, o_ref):
    s = jnp.sum(cs_ref[...], axis=0, keepdims=True) + jnp.sum(
        dl_ref[...], axis=0, keepdims=True)
    t = s * (1.0 / _MS)
    o_ref[0, 0] = jnp.sum(t * t) + float(_C * (_C - 1))


def _normalize_row(buf, i):
    vs = [buf[i, pl.ds(f * 16, 16)] for f in range(_NF)]
    ss = vs[0] * vs[0]
    for f in range(1, _NF):
        ss = ss + vs[f] * vs[f]
    s = jnp.sum(ss)
    sv = jnp.full((16,), s, jnp.float32)
    bits = plsc.bitcast(sv, jnp.int32)
    y = plsc.bitcast(jnp.int32(0x5F3759DF) - lax.shift_right_arithmetic(bits, 1),
                     jnp.float32)
    for _ in range(3):
        y = y * (1.5 - 0.5 * sv * y * y)
    y = jnp.where(sv > 0.0, y, 0.0)
    ws = []
    for f in range(_NF):
        w = vs[f] * y
        buf[i, pl.ds(f * 16, 16)] = w
        ws.append(w)
    return ws


def _plan_body(mem_h, idx_h, val_h, delta_h, rflat_h, jflat_h, ncnt_h, wdat_h,
               idx_v, pbuf, nb, rlist, jlist, stage_r, stage_j,
               vbuf, mbuf, acc, nbuf, spidx, sem_a, sem_b):
    cid = lax.axis_index("c")
    sid = lax.axis_index("s")
    wid = sid * 2 + cid
    lo = wid * _RW

    @pl.when(sid == 0)
    def _():
        pltpu.sync_copy(idx_h, spidx)
    plsc.subcore_barrier()
    pltpu.sync_copy(spidx, idx_v)

    lane = lax.iota(jnp.int32, 16)
    neg1 = jnp.full((16,), -1, jnp.int32)
    zeros = jnp.zeros((16,), jnp.float32)

    def _init(i, _):
        pbuf[pl.ds(i * 16, 16)] = neg1
        return 0
    lax.fori_loop(0, _PPAD // 16, _init, 0)
    nb[pl.ds(16, 16)] = neg1

    def _p1(i, _):
        iv = idx_v[pl.ds(i * 16, 16)]
        key = iv * 16 + lane
        sk, _sv = plsc.sort_key_val(key, key)
        sidx = lax.shift_right_arithmetic(sk, 4)
        nb[pl.ds(0, 16)] = sidx
        nxt = nb[pl.ds(1, 16)]
        keep = jnp.not_equal(sidx, nxt)
        offs = sidx - lo
        inr = (offs >= 0) & (offs < _RW)
        m = keep & inr
        offs_c = jnp.where(m, offs, 0)
        jwin = (sk & 15) + i * 16
        plsc.store_scatter(pbuf, [offs_c], jwin, mask=m)
        return 0
    lax.fori_loop(0, _N // 16, _p1, 0)

    def _p2(k, o):
        pv = pbuf[pl.ds(k * 16, 16)]
        msk = pv >= 0
        rvec = (lo + k * 16) + lane
        plsc.store_compressed(rlist.at[pl.ds(o, 16)], rvec, mask=msk)
        plsc.store_compressed(jlist.at[pl.ds(o, 16)], pv, mask=msk)
        return o + jnp.sum(msk.astype(jnp.int32))
    n = lax.fori_loop(0, _PPAD // 16, _p2, 0)

    nbuf[pl.ds(0, 16)] = jnp.full((16,), n, jnp.int32)
    pltpu.sync_copy(nbuf, ncnt_h.at[wid])

    for f in range(_NF):
        acc[pl.ds(f * 16, 16)] = zeros

    @pl.when(n > 0)
    def _():
        r0 = rlist[pl.ds(0, 16)][0]
        j0 = jlist[pl.ds(0, 16)][0]
        r0v = jnp.full((16,), r0, jnp.int32)
        j0v = jnp.full((16,), j0, jnp.int32)
        for t in range(_KC // 16):
            rlist[pl.ds(n + t * 16, 16)] = r0v
            jlist[pl.ds(n + t * 16, 16)] = j0v
        nch = (n + _KC - 1) // _KC

        def _p3(c, accs):
            base = c * _KC
            for t in range(_KC // 16):
                stage_r[pl.ds(t * 16, 16)] = rlist[pl.ds(base + t * 16, 16)]
                stage_j[pl.ds(t * 16, 16)] = jlist[pl.ds(base + t * 16, 16)]
            d1 = pltpu.async_copy(val_h.at[stage_j], vbuf, sem_a)
            d2 = pltpu.async_copy(mem_h.at[stage_r], mbuf, sem_b)
            pltpu.sync_copy(stage_r, rflat_h.at[wid, c])
            pltpu.sync_copy(stage_j, jflat_h.at[wid, c])
            d1.wait()
            d2.wait()

            def _row(i, a):
                ws = _normalize_row(vbuf, i)
                return tuple(
                    a[f] + (ws[f] - mbuf[i, pl.ds(f * 16, 16)])
                    for f in range(_NF))
            accs = lax.fori_loop(0, _KC, _row, accs)

            @pl.when(c < _WCAP)
            def _():
                pltpu.sync_copy(vbuf, wdat_h.at[wid, c])
            return accs

        accs = lax.fori_loop(0, nch, _p3, (zeros,) * _NF)

        npad = nch * _KC - n
        w = npad.astype(jnp.float32)
        for f in range(_NF):
            pad_dv = (vbuf[_KC - 1, pl.ds(f * 16, 16)]
                      - mbuf[_KC - 1, pl.ds(f * 16, 16)])
            acc[pl.ds(f * 16, 16)] = accs[f] - pad_dv * w

    pltpu.sync_copy(acc, delta_h.at[wid])


def _apply_body(val_h, rflat_h, jflat_h, ncnt_h, wdat_h, out_h,
                nbuf, stage_r, stage_j, vbuf, sem_a, sem_b):
    cid = lax.axis_index("c")
    sid = lax.axis_index("s")
    wid = sid * 2 + cid

    pltpu.async_copy(ncnt_h.at[wid], nbuf, sem_b)
    for b in range(_NB):
        pltpu.async_copy(rflat_h.at[wid, b], stage_r.at[b], sem_b)
        pltpu.async_copy(wdat_h.at[wid, b], vbuf.at[b], sem_a)
    pltpu.make_async_copy(ncnt_h.at[wid], nbuf, sem_b).wait()
    for b in range(_NB):
        pltpu.make_async_copy(rflat_h.at[wid, b], stage_r.at[b], sem_b).wait()
        pltpu.make_async_copy(wdat_h.at[wid, b], vbuf.at[b], sem_a).wait()
    n = nbuf[pl.ds(0, 16)][0]
    nch = (n + _KC - 1) // _KC

    for b in range(_NB):
        @pl.when(b < nch)
        def _():
            pltpu.async_copy(vbuf.at[b], out_h.at[stage_r.at[b]], sem_a)
    for b in range(_NB):
        @pl.when(b < nch)
        def _():
            pltpu.make_async_copy(vbuf.at[b], out_h.at[stage_r.at[b]],
                                  sem_a).wait()

    def _tail(c, _):
        pltpu.sync_copy(rflat_h.at[wid, c], stage_r.at[0])

        @pl.when(c < _WCAP)
        def _():
            pltpu.sync_copy(wdat_h.at[wid, c], vbuf.at[0])

        @pl.when(c >= _WCAP)
        def _():
            pltpu.sync_copy(jflat_h.at[wid, c], stage_j.at[0])
            pltpu.async_copy(val_h.at[stage_j.at[0]], vbuf.at[0],
                             sem_a).wait()

            def _row(i, _):
                _normalize_row(vbuf.at[0], i)
                return 0
            lax.fori_loop(0, _KC, _row, 0)

        pltpu.async_copy(vbuf.at[0], out_h.at[stage_r.at[0]], sem_a).wait()
        return 0
    lax.fori_loop(_NB, nch, _tail, 0)


_copy_call = pl.pallas_call(
    _copy_body,
    grid=(_M // _BR,),
    in_specs=[pl.BlockSpec((_BR, _D), lambda i: (i, 0))],
    out_specs=[
        pl.BlockSpec((_BR, _D), lambda i: (i, 0)),
        pl.BlockSpec((8, _D), lambda i: (0, 0)),
    ],
    out_shape=[
        jax.ShapeDtypeStruct((_M, _D), jnp.float32),
        jax.ShapeDtypeStruct((8, _D), jnp.float32),
    ],
)

_final_call = pl.pallas_call(
    _final_body,
    out_specs=pl.BlockSpec(memory_space=pltpu.SMEM),
    out_shape=jax.ShapeDtypeStruct((1, 1), jnp.float32),
)


@functools.cache
def _get_plan_call():
  return pl.kernel(
    _plan_body,
    out_type=(
        jax.ShapeDtypeStruct((_NW, _D), jnp.float32),
        jax.ShapeDtypeStruct((_NW, _NCH, _KC), jnp.int32),
        jax.ShapeDtypeStruct((_NW, _NCH, _KC), jnp.int32),
        jax.ShapeDtypeStruct((_NW, 16), jnp.int32),
        jax.ShapeDtypeStruct((_NW, _WCAP, _KC, _D), jnp.float32),
    ),
    mesh=plsc.VectorSubcoreMesh(core_axis_name="c", subcore_axis_name="s"),
    compiler_params=pltpu.CompilerParams(needs_layout_passes=False),
    scratch_types=[
        pltpu.VMEM((_N,), jnp.int32),
        pltpu.VMEM((_PPAD,), jnp.int32),
        pltpu.VMEM((32,), jnp.int32),
        pltpu.VMEM((_LCAP,), jnp.int32),
        pltpu.VMEM((_LCAP,), jnp.int32),
        pltpu.VMEM((_KC,), jnp.int32),
        pltpu.VMEM((_KC,), jnp.int32),
        pltpu.VMEM((_KC, _D), jnp.float32),
        pltpu.VMEM((_KC, _D), jnp.float32),
        pltpu.VMEM((_D,), jnp.float32),
        pltpu.VMEM((16,), jnp.int32),
        pltpu.VMEM_SHARED((_N,), jnp.int32),
        pltpu.SemaphoreType.DMA,
        pltpu.SemaphoreType.DMA,
    ],
  )


@functools.cache
def _get_apply_call():
  return pl.kernel(
    _apply_body,
    out_type=(),
    mesh=plsc.VectorSubcoreMesh(core_axis_name="c", subcore_axis_name="s"),
    compiler_params=pltpu.CompilerParams(needs_layout_passes=False),
    scratch_types=[
        pltpu.VMEM((16,), jnp.int32),
        pltpu.VMEM((_NB, _KC), jnp.int32),
        pltpu.VMEM((_NB, _KC), jnp.int32),
        pltpu.VMEM((_NB, _KC, _D), jnp.float32),
        pltpu.SemaphoreType.DMA,
        pltpu.SemaphoreType.DMA,
    ],
  )


@jax.jit
def kernel(mem, idx, val):
    idx = idx.astype(jnp.int32)
    delta, rflat, jflat, ncnt, wdat = _get_plan_call()(mem, idx, val)
    out_copy, colsum8 = _copy_call(mem)
    out_ref = jax.new_ref(out_copy)
    _get_apply_call()(val, rflat, jflat, ncnt, wdat, out_ref)
    ratio_arr = _final_call(colsum8, delta)
    out_final = jax.freeze(out_ref)
    return (out_final, ratio_arr[0, 0])


# speedup vs baseline: 1.0741x; 1.0066x over previous
import functools

import jax
import jax.numpy as jnp
from jax import lax
from jax.experimental import pallas as pl
from jax.experimental.pallas import tpu as pltpu
from jax.experimental.pallas import tpu_sc as plsc

_C = 19
_MS = 20000
_M = _C * _MS
_D = 128
_N = 16384

_NW = 32
_RW = _M // _NW
_PPAD = ((_RW + 15) // 16) * 16
_KC = 128
_NCH = _N // _KC + 1
_LCAP = _NCH * _KC
_WCAP = 8
_NB = 5

_BR = 20000
_NF = _D // 16


def _copy_body(m_ref, o_ref, s_ref):
    i = pl.program_id(0)
    blk = m_ref[...]
    o_ref[...] = blk
    part = blk.reshape(_BR // 8, 8, _D).sum(axis=0)

    @pl.when(i == 0)
    def _():
        s_ref[...] = part

    @pl.when(i > 0)
    def _():
        s_ref[...] = s_ref[...] + part


def _final_body(cs_ref, dl_ref, o_ref):
    s = jnp.sum(cs_ref[...], axis=0, keepdims=True) + jnp.sum(
        dl_ref[...], axis=0, keepdims=True)
    t = s * (1.0 / _MS)
    o_ref[0, 0] = jnp.sum(t * t) + float(_C * (_C - 1))


def _normalize_row(buf, i):
    vs = [buf[i, pl.ds(f * 16, 16)] for f in range(_NF)]
    ss = vs[0] * vs[0]
    for f in range(1, _NF):
        ss = ss + vs[f] * vs[f]
    s = jnp.sum(ss)
    sv = jnp.full((16,), s, jnp.float32)
    bits = plsc.bitcast(sv, jnp.int32)
    y = plsc.bitcast(jnp.int32(0x5F3759DF) - lax.shift_right_arithmetic(bits, 1),
                     jnp.float32)
    for _ in range(3):
        y = y * (1.5 - 0.5 * sv * y * y)
    y = jnp.where(sv > 0.0, y, 0.0)
    ws = []
    for f in range(_NF):
        w = vs[f] * y
        buf[i, pl.ds(f * 16, 16)] = w
        ws.append(w)
    return ws


def _plan_body(mem_h, idx_h, val_h, delta_h, rflat_h, jflat_h, ncnt_h, wdat_h,
               idx_v, pbuf, nb, rlist, jlist, stage_r, stage_j,
               vbuf, mbuf, acc, nbuf, spidx, sem_a, sem_b):
    cid = lax.axis_index("c")
    sid = lax.axis_index("s")
    wid = sid * 2 + cid
    lo = wid * _RW

    @pl.when(sid == 0)
    def _():
        pltpu.sync_copy(idx_h, spidx)
    plsc.subcore_barrier()
    pltpu.sync_copy(spidx, idx_v)

    lane = lax.iota(jnp.int32, 16)
    neg1 = jnp.full((16,), -1, jnp.int32)
    zeros = jnp.zeros((16,), jnp.float32)

    def _init(i, _):
        pbuf[pl.ds(i * 16, 16)] = neg1
        return 0
    lax.fori_loop(0, _PPAD // 16, _init, 0)
    nb[pl.ds(16, 16)] = neg1

    def _p1(i, _):
        iv = idx_v[pl.ds(i * 16, 16)]
        key = iv * 16 + lane
        sk, _sv = plsc.sort_key_val(key, key)
        sidx = lax.shift_right_arithmetic(sk, 4)
        nb[pl.ds(0, 16)] = sidx
        nxt = nb[pl.ds(1, 16)]
        keep = jnp.not_equal(sidx, nxt)
        offs = sidx - lo
        inr = (offs >= 0) & (offs < _RW)
        m = keep & inr
        offs_c = jnp.where(m, offs, 0)
        jwin = (sk & 15) + i * 16
        plsc.store_scatter(pbuf, [offs_c], jwin, mask=m)
        return 0
    lax.fori_loop(0, _N // 16, _p1, 0)

    def _p2(k, o):
        pv = pbuf[pl.ds(k * 16, 16)]
        msk = pv >= 0
        rvec = (lo + k * 16) + lane
        plsc.store_compressed(rlist.at[pl.ds(o, 16)], rvec, mask=msk)
        plsc.store_compressed(jlist.at[pl.ds(o, 16)], pv, mask=msk)
        return o + jnp.sum(msk.astype(jnp.int32))
    n = lax.fori_loop(0, _PPAD // 16, _p2, 0)

    nbuf[pl.ds(0, 16)] = jnp.full((16,), n, jnp.int32)
    pltpu.sync_copy(nbuf, ncnt_h.at[wid])

    for f in range(_NF):
        acc[pl.ds(f * 16, 16)] = zeros

    @pl.when(n > 0)
    def _():
        r0 = rlist[pl.ds(0, 16)][0]
        j0 = jlist[pl.ds(0, 16)][0]
        r0v = jnp.full((16,), r0, jnp.int32)
        j0v = jnp.full((16,), j0, jnp.int32)
        for t in range(_KC // 16):
            rlist[pl.ds(n + t * 16, 16)] = r0v
            jlist[pl.ds(n + t * 16, 16)] = j0v
        nch = (n + _KC - 1) // _KC

        def _p3(c, accs):
            base = c * _KC
            for t in range(_KC // 16):
                stage_r[pl.ds(t * 16, 16)] = rlist[pl.ds(base + t * 16, 16)]
                stage_j[pl.ds(t * 16, 16)] = jlist[pl.ds(base + t * 16, 16)]
            d1 = pltpu.async_copy(val_h.at[stage_j], vbuf, sem_a)
            d2 = pltpu.async_copy(mem_h.at[stage_r], mbuf, sem_b)
            pltpu.sync_copy(stage_r, rflat_h.at[wid, c])
            pltpu.sync_copy(stage_j, jflat_h.at[wid, c])
            d1.wait()
            d2.wait()

            def _row(i, a):
                ws = _normalize_row(vbuf, i)
                return tuple(
                    a[f] + (ws[f] - mbuf[i, pl.ds(f * 16, 16)])
                    for f in range(_NF))
            accs = lax.fori_loop(0, _KC, _row, accs)

            @pl.when(c < _WCAP)
            def _():
                pltpu.sync_copy(vbuf, wdat_h.at[wid, c])
            return accs

        accs = lax.fori_loop(0, nch, _p3, (zeros,) * _NF)

        npad = nch * _KC - n
        w = npad.astype(jnp.float32)
        for f in range(_NF):
            pad_dv = (vbuf[_KC - 1, pl.ds(f * 16, 16)]
                      - mbuf[_KC - 1, pl.ds(f * 16, 16)])
            acc[pl.ds(f * 16, 16)] = accs[f] - pad_dv * w

    pltpu.sync_copy(acc, delta_h.at[wid])


def _apply_body(val_h, rflat_h, jflat_h, ncnt_h, wdat_h, out_h,
                nbuf, stage_r, stage_j, vbuf, sem_a, sem_b):
    cid = lax.axis_index("c")
    sid = lax.axis_index("s")
    wid = sid * 2 + cid

    pltpu.async_copy(ncnt_h.at[wid], nbuf, sem_b)
    for b in range(_NB):
        pltpu.async_copy(rflat_h.at[wid, b], stage_r.at[b], sem_b)
        pltpu.async_copy(wdat_h.at[wid, b], vbuf.at[b], sem_a)
    pltpu.make_async_copy(ncnt_h.at[wid], nbuf, sem_b).wait()
    for b in range(_NB):
        pltpu.make_async_copy(rflat_h.at[wid, b], stage_r.at[b], sem_b).wait()
        pltpu.make_async_copy(wdat_h.at[wid, b], vbuf.at[b], sem_a).wait()
    n = nbuf[pl.ds(0, 16)][0]
    nch = (n + _KC - 1) // _KC

    for b in range(_NB):
        @pl.when(b < nch)
        def _():
            pltpu.async_copy(vbuf.at[b], out_h.at[stage_r.at[b]], sem_a)
    for b in range(_NB):
        @pl.when(b < nch)
        def _():
            pltpu.make_async_copy(vbuf.at[b], out_h.at[stage_r.at[b]],
                                  sem_a).wait()

    def _tail(c, _):
        pltpu.sync_copy(rflat_h.at[wid, c], stage_r.at[0])

        @pl.when(c < _WCAP)
        def _():
            pltpu.sync_copy(wdat_h.at[wid, c], vbuf.at[0])

        @pl.when(c >= _WCAP)
        def _():
            pltpu.sync_copy(jflat_h.at[wid, c], stage_j.at[0])
            pltpu.async_copy(val_h.at[stage_j.at[0]], vbuf.at[0],
                             sem_a).wait()

            def _row(i, _):
                _normalize_row(vbuf.at[0], i)
                return 0
            lax.fori_loop(0, _KC, _row, 0)

        pltpu.async_copy(vbuf.at[0], out_h.at[stage_r.at[0]], sem_a).wait()
        return 0
    lax.fori_loop(_NB, nch, _tail, 0)


_copy_call = pl.pallas_call(
    _copy_body,
    grid=(_M // _BR,),
    in_specs=[pl.BlockSpec((_BR, _D), lambda i: (i, 0))],
    out_specs=[
        pl.BlockSpec((_BR, _D), lambda i: (i, 0)),
        pl.BlockSpec((8, _D), lambda i: (0, 0)),
    ],
    out_shape=[
        jax.ShapeDtypeStruct((_M, _D), jnp.float32),
        jax.ShapeDtypeStruct((8, _D), jnp.float32),
    ],
)

_final_call = pl.pallas_call(
    _final_body,
    out_specs=pl.BlockSpec(memory_space=pltpu.SMEM),
    out_shape=jax.ShapeDtypeStruct((1, 1), jnp.float32),
)


@functools.cache
def _get_plan_call():
  return pl.kernel(
    _plan_body,
    out_type=(
        jax.ShapeDtypeStruct((_NW, _D), jnp.float32),
        jax.ShapeDtypeStruct((_NW, _NCH, _KC), jnp.int32),
        jax.ShapeDtypeStruct((_NW, _NCH, _KC), jnp.int32),
        jax.ShapeDtypeStruct((_NW, 16), jnp.int32),
        jax.ShapeDtypeStruct((_NW, _WCAP, _KC, _D), jnp.float32),
    ),
    mesh=plsc.VectorSubcoreMesh(core_axis_name="c", subcore_axis_name="s"),
    compiler_params=pltpu.CompilerParams(needs_layout_passes=False),
    scratch_types=[
        pltpu.VMEM((_N,), jnp.int32),
        pltpu.VMEM((_PPAD,), jnp.int32),
        pltpu.VMEM((32,), jnp.int32),
        pltpu.VMEM((_LCAP,), jnp.int32),
        pltpu.VMEM((_LCAP,), jnp.int32),
        pltpu.VMEM((_KC,), jnp.int32),
        pltpu.VMEM((_KC,), jnp.int32),
        pltpu.VMEM((_KC, _D), jnp.float32),
        pltpu.VMEM((_KC, _D), jnp.float32),
        pltpu.VMEM((_D,), jnp.float32),
        pltpu.VMEM((16,), jnp.int32),
        pltpu.VMEM_SHARED((_N,), jnp.int32),
        pltpu.SemaphoreType.DMA,
        pltpu.SemaphoreType.DMA,
    ],
  )


@functools.cache
def _get_apply_call():
  return pl.kernel(
    _apply_body,
    out_type=(),
    mesh=plsc.VectorSubcoreMesh(core_axis_name="c", subcore_axis_name="s"),
    compiler_params=pltpu.CompilerParams(needs_layout_passes=False),
    scratch_types=[
        pltpu.VMEM((16,), jnp.int32),
        pltpu.VMEM((_NB, _KC), jnp.int32),
        pltpu.VMEM((_NB, _KC), jnp.int32),
        pltpu.VMEM((_NB, _KC, _D), jnp.float32),
        pltpu.SemaphoreType.DMA,
        pltpu.SemaphoreType.DMA,
    ],
  )


@jax.jit
def kernel(mem, idx, val):
    idx = idx.astype(jnp.int32)
    delta, rflat, jflat, ncnt, wdat = _get_plan_call()(mem, idx, val)
    out_copy, colsum8 = _copy_call(mem)
    out_ref = jax.new_ref(out_copy)
    _get_apply_call()(val, rflat, jflat, ncnt, wdat, out_ref)
    ratio_arr = _final_call(colsum8, delta)
    out_final = jax.freeze(out_ref)
    return (out_final, ratio_arr[0, 0])
